# Initial kernel scaffold; baseline (speedup 1.0000x reference)
import jax
import jax.numpy as jnp
from jax.experimental import pallas as pl


def kernel(up_output, left_output, right_output):
    raise NotImplementedError("write your pallas kernel here")


# speedup vs baseline: 7.4258x; 7.4258x over previous
import functools

import jax
import jax.numpy as jnp
from jax import lax
from jax.experimental import pallas as pl
from jax.experimental.pallas import tpu as pltpu
from jax.experimental.pallas import tpu_sc as plsc

_COL0 = 110
_NBINS = 64


def _sc_scatter_kernel(up_hbm, out_l, out_r, up_v, acc_v):
    c = lax.axis_index("c")
    s = lax.axis_index("s")
    j0 = s * 16
    i0 = c * 128
    pltpu.sync_copy(up_hbm.at[pl.ds(i0, 128), pl.ds(j0, 16)], up_v)

    zero16 = jnp.zeros((16,), jnp.float32)
    for r in range(16):
        for q in range(_NBINS // 16):
            acc_v[r, pl.ds(q * 16, 16)] = zero16

    rows = lax.iota(jnp.int32, 16)
    is_left = c == 0

    def body(t, carry):
        ridx = jnp.where(is_left, 127 - t, t)
        v = up_v[ridx, :]
        m = v >= 0.0235
        col = (v * 50.0 + 110.5).astype(jnp.int32) - _COL0
        col = jnp.minimum(jnp.maximum(col, 0), _NBINS - 1)
        vf = (t + jnp.where(is_left, 1, 0)).astype(jnp.float32) * (1.0 / 60.0)
        val = jnp.full((16,), vf, jnp.float32)
        plsc.store_scatter(acc_v, [rows, col], val, mask=m)
        return carry

    lax.fori_loop(0, 128, body, 0)

    @pl.when(is_left)
    def _():
        pltpu.sync_copy(acc_v, out_l.at[pl.ds(j0, 16), :])

    @pl.when(jnp.logical_not(is_left))
    def _():
        pltpu.sync_copy(acc_v, out_r.at[pl.ds(j0, 16), :])


_sc_scatter = functools.partial(
    pl.kernel,
    mesh=plsc.VectorSubcoreMesh(core_axis_name="c", subcore_axis_name="s"),
    out_type=[
        jax.ShapeDtypeStruct((256, _NBINS), jnp.float32),
        jax.ShapeDtypeStruct((256, _NBINS), jnp.float32),
    ],
    scratch_types=[
        pltpu.VMEM((128, 16), jnp.float32),
        pltpu.VMEM((16, _NBINS), jnp.float32),
    ],
    compiler_params=pltpu.CompilerParams(
        use_tc_tiling_on_sc=False, needs_layout_passes=False
    ),
)(_sc_scatter_kernel)


def _loss_kernel(u2l_ref, u2r_ref, l_ref, r_ref, out_ref):
    threshold = 0.2
    a = u2l_ref[:]
    d = jnp.abs(a - l_ref[:])
    sl = jnp.sum(jnp.where((d < threshold) & (a != 0.0), d, 0.0))
    b = u2r_ref[:]
    d2 = jnp.abs(b - r_ref[:])
    sr = jnp.sum(jnp.where((d2 < threshold) & (b != 0.0), d2, 0.0))
    out_ref[0, 0] = (sl + sr) * (1.0 / 65536.0)


@jax.jit
def kernel(up_output, left_output, right_output):
    up = up_output.reshape(256, 256)
    l64 = left_output.reshape(256, 256)[:, _COL0:_COL0 + _NBINS]
    r64 = right_output.reshape(256, 256)[:, _COL0:_COL0 + _NBINS]
    u2l, u2r = _sc_scatter(up)
    loss = pl.pallas_call(
        _loss_kernel,
        out_shape=jax.ShapeDtypeStruct((1, 1), jnp.float32),
        out_specs=pl.BlockSpec(memory_space=pltpu.SMEM),
    )(u2l, u2r, l64, r64)
    return loss[0, 0]


# speedup vs baseline: 7.7385x; 1.0421x over previous
import functools

import jax
import jax.numpy as jnp
from jax import lax
from jax.experimental import pallas as pl
from jax.experimental.pallas import tpu as pltpu
from jax.experimental.pallas import tpu_sc as plsc

_COL0 = 110
_NBINS = 64


_MAGICF = 8388608.0 + 110.0
_MAGICI = -(0x4B000000 + _COL0)
_K = 4
_SUB = 128 // _K


def _sc_scatter_kernel(up_hbm, out_l, out_r, up_v, a0, a1, a2, a3, sem):
    accs = (a0, a1, a2, a3)
    c = lax.axis_index("c")
    s = lax.axis_index("s")
    j0 = s * 16
    i0 = c * 128
    copy = pltpu.async_copy(up_hbm.at[pl.ds(i0, 128), pl.ds(j0, 16)], up_v, sem)

    zero16 = jnp.zeros((16,), jnp.float32)

    def zbody(r, carry):
        for a in accs:
            for q in range(_NBINS // 16):
                a[r, pl.ds(q * 16, 16)] = zero16
        return carry

    lax.fori_loop(0, 16, zbody, 0)
    copy.wait()

    rows = lax.iota(jnp.int32, 16)
    is_left = c == 0

    def body(t, carry):
        tl = jnp.where(is_left, _SUB - 1 - t, t)
        es = [k * _SUB + tl for k in range(_K)]
        vs = [up_v[e, :] for e in es]
        ms = [v >= 0.0235 for v in vs]
        cols = [
            plsc.bitcast(v * 50.0 + _MAGICF, jnp.int32) + _MAGICI for v in vs
        ]
        vfs = [
            jnp.where(is_left, 128 - e, e).astype(jnp.float32) * (1.0 / 60.0)
            for e in es
        ]
        for k in range(_K):
            val = jnp.full((16,), vfs[k], jnp.float32)
            plsc.store_scatter(accs[k], [rows, cols[k]], val, mask=ms[k])
        return carry

    lax.fori_loop(0, _SUB, body, 0)

    def mbody(r, carry):
        for q in range(_NBINS // 16):
            sl = pl.ds(q * 16, 16)
            m01 = jnp.maximum(accs[0][r, sl], accs[1][r, sl])
            m23 = jnp.maximum(accs[2][r, sl], accs[3][r, sl])
            accs[0][r, sl] = jnp.maximum(m01, m23)
        return carry

    lax.fori_loop(0, 16, mbody, 0)

    @pl.when(is_left)
    def _():
        pltpu.sync_copy(accs[0], out_l.at[pl.ds(j0, 16), :])

    @pl.when(jnp.logical_not(is_left))
    def _():
        pltpu.sync_copy(accs[0], out_r.at[pl.ds(j0, 16), :])


_sc_scatter = functools.partial(
    pl.kernel,
    mesh=plsc.VectorSubcoreMesh(core_axis_name="c", subcore_axis_name="s"),
    out_type=[
        jax.ShapeDtypeStruct((256, _NBINS), jnp.float32),
        jax.ShapeDtypeStruct((256, _NBINS), jnp.float32),
    ],
    scratch_types=[
        pltpu.VMEM((128, 16), jnp.float32),
        pltpu.VMEM((16, _NBINS), jnp.float32),
        pltpu.VMEM((16, _NBINS), jnp.float32),
        pltpu.VMEM((16, _NBINS), jnp.float32),
        pltpu.VMEM((16, _NBINS), jnp.float32),
        pltpu.SemaphoreType.DMA,
    ],
    compiler_params=pltpu.CompilerParams(
        use_tc_tiling_on_sc=False, needs_layout_passes=False
    ),
)(_sc_scatter_kernel)


def _loss_kernel(u2l_ref, u2r_ref, l_ref, r_ref, out_ref):
    threshold = 0.2
    a = u2l_ref[:]
    d = jnp.abs(a - l_ref[:])
    sl = jnp.sum(jnp.where((d < threshold) & (a != 0.0), d, 0.0))
    b = u2r_ref[:]
    d2 = jnp.abs(b - r_ref[:])
    sr = jnp.sum(jnp.where((d2 < threshold) & (b != 0.0), d2, 0.0))
    out_ref[0, 0] = (sl + sr) * (1.0 / 65536.0)


@jax.jit
def kernel(up_output, left_output, right_output):
    up = up_output.reshape(256, 256)
    l64 = left_output.reshape(256, 256)[:, _COL0:_COL0 + _NBINS]
    r64 = right_output.reshape(256, 256)[:, _COL0:_COL0 + _NBINS]
    u2l, u2r = _sc_scatter(up)
    loss = pl.pallas_call(
        _loss_kernel,
        out_shape=jax.ShapeDtypeStruct((1, 1), jnp.float32),
        out_specs=pl.BlockSpec(memory_space=pltpu.SMEM),
    )(u2l, u2r, l64, r64)
    return loss[0, 0]


# speedup vs baseline: 7.9882x; 1.0323x over previous
import functools

import jax
import jax.numpy as jnp
from jax import lax
from jax.experimental import pallas as pl
from jax.experimental.pallas import tpu as pltpu
from jax.experimental.pallas import tpu_sc as plsc

_COL0 = 110
_NBINS = 128
_MERGEQ = 4


_MAGICF = 8388608.0 + 110.0
_MAGICI = -(0x4B000000 + _COL0)
_K = 4
_SUB = 128 // _K


def _sc_scatter_kernel(up_hbm, out_l, out_r, up_v, a0, a1, a2, a3, sem):
    accs = (a0, a1, a2, a3)
    c = lax.axis_index("c")
    s = lax.axis_index("s")
    j0 = s * 16
    i0 = c * 128
    jb = s // 8
    q16 = (s % 8) * 16
    copy = pltpu.async_copy(
        up_hbm.at[pl.ds(i0, 128), pl.ds(jb * 128, 128)], up_v, sem
    )

    zero16 = jnp.zeros((16,), jnp.float32)

    def zbody(r, carry):
        for a in accs:
            for q in range(_NBINS // 16):
                a[r, pl.ds(q * 16, 16)] = zero16
        return carry

    lax.fori_loop(0, 16, zbody, 0)
    copy.wait()

    rows = lax.iota(jnp.int32, 16)
    is_left = c == 0

    def body(t, carry):
        tl = jnp.where(is_left, _SUB - 1 - t, t)
        es = [k * _SUB + tl for k in range(_K)]
        vs = [up_v[e, pl.ds(q16, 16)] for e in es]
        ms = [v >= 0.0235 for v in vs]
        cols = [
            plsc.bitcast(v * 50.0 + _MAGICF, jnp.int32) + _MAGICI for v in vs
        ]
        vfs = [
            jnp.where(is_left, 128 - e, e).astype(jnp.float32) * (1.0 / 60.0)
            for e in es
        ]
        for k in range(_K):
            val = jnp.full((16,), vfs[k], jnp.float32)
            plsc.store_scatter(accs[k], [rows, cols[k]], val, mask=ms[k])
        return carry

    lax.fori_loop(0, _SUB, body, 0)

    def mbody(r, carry):
        for q in range(_MERGEQ):
            sl = pl.ds(q * 16, 16)
            m01 = jnp.maximum(accs[0][r, sl], accs[1][r, sl])
            m23 = jnp.maximum(accs[2][r, sl], accs[3][r, sl])
            accs[0][r, sl] = jnp.maximum(m01, m23)
        return carry

    lax.fori_loop(0, 16, mbody, 0)

    @pl.when(is_left)
    def _():
        pltpu.sync_copy(accs[0], out_l.at[pl.ds(j0, 16), :])

    @pl.when(jnp.logical_not(is_left))
    def _():
        pltpu.sync_copy(accs[0], out_r.at[pl.ds(j0, 16), :])


_sc_scatter = functools.partial(
    pl.kernel,
    mesh=plsc.VectorSubcoreMesh(core_axis_name="c", subcore_axis_name="s"),
    out_type=[
        jax.ShapeDtypeStruct((256, _NBINS), jnp.float32),
        jax.ShapeDtypeStruct((256, _NBINS), jnp.float32),
    ],
    scratch_types=[
        pltpu.VMEM((128, 128), jnp.float32),
        pltpu.VMEM((16, _NBINS), jnp.float32),
        pltpu.VMEM((16, _NBINS), jnp.float32),
        pltpu.VMEM((16, _NBINS), jnp.float32),
        pltpu.VMEM((16, _NBINS), jnp.float32),
        pltpu.SemaphoreType.DMA,
    ],
    compiler_params=pltpu.CompilerParams(
        use_tc_tiling_on_sc=True, needs_layout_passes=False
    ),
)(_sc_scatter_kernel)


def _loss_kernel(u2l_ref, u2r_ref, l_ref, r_ref, out_ref):
    threshold = 0.2
    a = u2l_ref[:]
    d = jnp.abs(a - l_ref[:])
    sl = jnp.sum(jnp.where((d < threshold) & (a != 0.0), d, 0.0))
    b = u2r_ref[:]
    d2 = jnp.abs(b - r_ref[:])
    sr = jnp.sum(jnp.where((d2 < threshold) & (b != 0.0), d2, 0.0))
    out_ref[0, 0] = (sl + sr) * (1.0 / 65536.0)


@jax.jit
def kernel(up_output, left_output, right_output):
    up = up_output.reshape(256, 256)
    l64 = left_output.reshape(256, 256)[:, _COL0:_COL0 + _NBINS]
    r64 = right_output.reshape(256, 256)[:, _COL0:_COL0 + _NBINS]
    u2l, u2r = _sc_scatter(up)
    loss = pl.pallas_call(
        _loss_kernel,
        out_shape=jax.ShapeDtypeStruct((1, 1), jnp.float32),
        out_specs=pl.BlockSpec(memory_space=pltpu.SMEM),
    )(u2l, u2r, l64, r64)
    return loss[0, 0]


# speedup vs baseline: 8.0202x; 1.0040x over previous
import functools

import jax
import jax.numpy as jnp
from jax import lax
from jax.experimental import pallas as pl
from jax.experimental.pallas import tpu as pltpu
from jax.experimental.pallas import tpu_sc as plsc

_COL0 = 110
_NBINS = 128
_MERGEQ = 4


_MAGICF = 8388608.0 + 110.0
_MAGICI = -(0x4B000000 + _COL0)
_K = 4
_SUB = 128 // _K


def _sc_scatter_kernel(up_hbm, out_l, out_r, up_v, a0, a1, a2, a3, sem):
    accs = (a0, a1, a2, a3)
    c = lax.axis_index("c")
    s = lax.axis_index("s")
    j0 = s * 16
    i0 = c * 128
    jb = s // 8
    q16 = (s % 8) * 16
    copy = pltpu.async_copy(
        up_hbm.at[pl.ds(i0, 128), pl.ds(jb * 128, 128)], up_v, sem
    )

    zero16 = jnp.zeros((16,), jnp.float32)

    def zbody(r, carry):
        for a in accs:
            for q in range(_NBINS // 16):
                a[r, pl.ds(q * 16, 16)] = zero16
        return carry

    lax.fori_loop(0, 16, zbody, 0)
    copy.wait()

    rows = lax.iota(jnp.int32, 16)
    is_left = c == 0

    def body(t, carry):
        tl = jnp.where(is_left, _SUB - 1 - t, t)
        es = [k * _SUB + tl for k in range(_K)]
        vs = [up_v[e, pl.ds(q16, 16)] for e in es]
        ms = [v >= 0.0235 for v in vs]
        cols = [
            plsc.bitcast(v * 50.0 + _MAGICF, jnp.int32) + _MAGICI for v in vs
        ]
        vfs = [
            jnp.where(is_left, 128 - e, e).astype(jnp.float32) * (1.0 / 60.0)
            for e in es
        ]
        for k in range(_K):
            val = jnp.full((16,), vfs[k], jnp.float32)
            plsc.store_scatter(accs[k], [rows, cols[k]], val, mask=ms[k])
        return carry

    lax.fori_loop(0, _SUB, body, 0)

    def mbody(r, carry):
        for q in range(_MERGEQ):
            sl = pl.ds(q * 16, 16)
            m01 = jnp.maximum(accs[0][r, sl], accs[1][r, sl])
            m23 = jnp.maximum(accs[2][r, sl], accs[3][r, sl])
            accs[0][r, sl] = jnp.maximum(m01, m23)
        return carry

    lax.fori_loop(0, 16, mbody, 0)

    @pl.when(is_left)
    def _():
        pltpu.sync_copy(accs[0], out_l.at[pl.ds(j0, 16), :])

    @pl.when(jnp.logical_not(is_left))
    def _():
        pltpu.sync_copy(accs[0], out_r.at[pl.ds(j0, 16), :])


_sc_scatter = functools.partial(
    pl.kernel,
    mesh=plsc.VectorSubcoreMesh(core_axis_name="c", subcore_axis_name="s"),
    out_type=[
        jax.ShapeDtypeStruct((256, _NBINS), jnp.float32),
        jax.ShapeDtypeStruct((256, _NBINS), jnp.float32),
    ],
    scratch_types=[
        pltpu.VMEM((128, 128), jnp.float32),
        pltpu.VMEM((16, _NBINS), jnp.float32),
        pltpu.VMEM((16, _NBINS), jnp.float32),
        pltpu.VMEM((16, _NBINS), jnp.float32),
        pltpu.VMEM((16, _NBINS), jnp.float32),
        pltpu.SemaphoreType.DMA,
    ],
    compiler_params=pltpu.CompilerParams(
        use_tc_tiling_on_sc=True,
        needs_layout_passes=False,
        skip_device_barrier=True,
    ),
)(_sc_scatter_kernel)


def _loss_kernel(u2l_ref, u2r_ref, l_ref, r_ref, out_ref):
    threshold = 0.2
    a = u2l_ref[:]
    d = jnp.abs(a - l_ref[:])
    sl = jnp.sum(jnp.where((d < threshold) & (a != 0.0), d, 0.0))
    b = u2r_ref[:]
    d2 = jnp.abs(b - r_ref[:])
    sr = jnp.sum(jnp.where((d2 < threshold) & (b != 0.0), d2, 0.0))
    out_ref[0, 0] = (sl + sr) * (1.0 / 65536.0)


@jax.jit
def kernel(up_output, left_output, right_output):
    up = up_output.reshape(256, 256)
    l64 = left_output.reshape(256, 256)[:, _COL0:_COL0 + _NBINS]
    r64 = right_output.reshape(256, 256)[:, _COL0:_COL0 + _NBINS]
    u2l, u2r = _sc_scatter(up)
    loss = pl.pallas_call(
        _loss_kernel,
        out_shape=jax.ShapeDtypeStruct((1, 1), jnp.float32),
        out_specs=pl.BlockSpec(memory_space=pltpu.SMEM),
    )(u2l, u2r, l64, r64)
    return loss[0, 0]
